# baseline (speedup 1.0000x reference)
import functools

import jax
import jax.numpy as jnp
from jax.experimental import pallas as pl
from jax.experimental.pallas import tpu as pltpu
from jax.experimental.pallas import tpu_sc as plsc

_N = 4096
_F = 1024
_TA = 1024
_TB = 256
_K = 3

_P = jax.lax.Precision.DEFAULT
_HI = jax.lax.Precision.HIGHEST


def _stage_ab1(p_ref, w1t_ref, b1_ref, w2t_ref, b2_ref, w3t_ref, b3_ref,
               feat_ref, gmax_ref, idx_ref, dk_ref, x_s, xxc_s, xxt_s):
    i = pl.program_id(0)

    @pl.when(i == 0)
    def _():
        ones = jnp.ones((1, _F), jnp.float32)
        for c in range(_N // _TA):
            sl = pl.ds(c * _TA, _TA)
            p = p_ref[sl, :]
            f1 = jax.nn.relu(
                jnp.dot(p, w1t_ref[...], precision=_P) + b1_ref[...])
            f = jax.nn.relu(
                jnp.dot(f1, w2t_ref[...], precision=_P) + b2_ref[...])
            f2 = jax.nn.relu(
                jnp.dot(f, w3t_ref[...], precision=_P) + b3_ref[...])
            feat_ref[sl, :] = f
            x_s[sl, :] = f2
            y = f2 * f2
            xxc_s[sl, :] = jnp.sum(y, axis=1, keepdims=True)
            xxt_s[:, sl] = jax.lax.dot_general(
                ones, y, (((1,), (1,)), ((), ())), precision=_HI)
            m = jnp.max(f2, axis=0, keepdims=True)
            if c == 0:
                gmax_ref[...] = m
            else:
                gmax_ref[...] = jnp.maximum(gmax_ref[...], m)

    r0 = i * _TB
    x_t2 = x_s[pl.ds(r0, _TB), :] * jnp.float32(-2.0)
    dots2 = jax.lax.dot_general(
        x_t2, x_s[...], (((1,), (1,)), ((), ())), precision=_P)
    t = (xxc_s[pl.ds(r0, _TB), :] + dots2) + xxt_s[...]

    iota = jax.lax.broadcasted_iota(jnp.int32, (_TB, _N), 1)
    vals = t
    idxs = []
    negd = []
    for r in range(_K + 1):
        m = jnp.max(vals, axis=1, keepdims=True)
        selmask = vals == m
        if r > 0:
            idxr = jnp.min(
                jnp.where(selmask, iota, _N), axis=1, keepdims=True)
            idxs.append(idxr)
            negd.append(-m)
        if r < _K:
            vals = jnp.where(selmask, jnp.float32(-1e30), vals)
    idx_ref[...] = jnp.concatenate(idxs, axis=1)
    dk_ref[...] = jnp.concatenate(negd, axis=1)


def _make_sc_gather():
    nc, ns = 2, 16
    nw = nc * ns
    b_total = _N * _K
    b_per_w = b_total // nw
    mesh = plsc.VectorSubcoreMesh(
        core_axis_name="c", subcore_axis_name="s",
        num_cores=nc, num_subcores=ns)

    @functools.partial(
        pl.kernel, mesh=mesh,
        out_type=jax.ShapeDtypeStruct((b_total, 16), jnp.float32),
        compiler_params=pltpu.CompilerParams(use_tc_tiling_on_sc=False),
        scratch_types=[
            pltpu.VMEM((b_per_w,), jnp.int32),
            pltpu.VMEM((b_per_w, 16), jnp.float32),
            pltpu.SemaphoreType.DMA,
        ],
    )
    def sc_gather(table_hbm, idx_hbm, out_hbm, idx_v, rows_v, sem):
        wid = jax.lax.axis_index("s") * nc + jax.lax.axis_index("c")
        base = wid * b_per_w
        pltpu.sync_copy(idx_hbm.at[pl.ds(base, b_per_w)], idx_v)
        copies = [
            pltpu.async_copy(
                table_hbm.at[idx_v.at[pl.ds(j * 128, 128)]],
                rows_v.at[pl.ds(j * 128, 128), :], sem)
            for j in range(b_per_w // 128)
        ]
        for c in copies:
            c.wait()
        pltpu.sync_copy(rows_v, out_hbm.at[pl.ds(base, b_per_w)])

    return sc_gather


def _stage_b2(feat_ref, knn_ref, dk_ref, pts_ref, gmax_ref,
              w4at_ref, b4_ref, w4bt_ref, w4ct_ref, w5t_ref, b5_ref,
              w6t_ref, b6_ref, out_ref, b4eff_ref):
    i = pl.program_id(0)

    @pl.when(i == 0)
    def _():
        b4eff_ref[...] = (
            jnp.dot(gmax_ref[...], w4at_ref[...], precision=_HI)
            + b4_ref[...])

    p_t = pts_ref[...]
    parts = []
    for r in range(_K):
        rel = knn_ref[:, r * 16:r * 16 + 3] - p_t
        parts.append(rel)
        parts.append(dk_ref[:, r:r + 1])
    knn12 = jnp.concatenate(parts, axis=1)

    h = jax.nn.relu(
        jnp.dot(feat_ref[...], w4bt_ref[...], precision=_P)
        + jnp.dot(knn12, w4ct_ref[...], precision=_HI)
        + b4eff_ref[...])
    h2 = jax.nn.relu(jnp.dot(h, w5t_ref[...], precision=_P) + b5_ref[...])
    logits = jnp.dot(h2, w6t_ref[...], precision=_P) + b6_ref[...]
    out_ref[...] = jax.nn.softmax(logits, axis=-1)


def kernel(points, W1, b1, W2, b2, W3, b3, W4, b4, W5, b5, W6, b6):
    pts = points.reshape(_N, 3)
    w1t, w2t, w3t = W1.T, W2.T, W3.T
    w4at = W4[:, :_F].T
    w4bt = W4[:, _F:_F + 64].T
    w4ct = W4[:, _F + 64:].T
    w5t, w6t = W5.T, W6.T
    b1r, b2r, b3r = b1.reshape(1, -1), b2.reshape(1, -1), b3.reshape(1, -1)
    b4r, b5r, b6r = b4.reshape(1, -1), b5.reshape(1, -1), b6.reshape(1, -1)

    nb = _N // _TB
    feat, gmax, idx, dk = pl.pallas_call(
        _stage_ab1,
        grid=(nb,),
        in_specs=[
            pl.BlockSpec((_N, 3), lambda i: (0, 0)),
            pl.BlockSpec((3, 64), lambda i: (0, 0)),
            pl.BlockSpec((1, 64), lambda i: (0, 0)),
            pl.BlockSpec((64, 64), lambda i: (0, 0)),
            pl.BlockSpec((1, 64), lambda i: (0, 0)),
            pl.BlockSpec((64, _F), lambda i: (0, 0)),
            pl.BlockSpec((1, _F), lambda i: (0, 0)),
        ],
        out_specs=[
            pl.BlockSpec((_N, 64), lambda i: (0, 0)),
            pl.BlockSpec((1, _F), lambda i: (0, 0)),
            pl.BlockSpec((_TB, _K), lambda i: (i, 0)),
            pl.BlockSpec((_TB, _K), lambda i: (i, 0)),
        ],
        out_shape=[
            jax.ShapeDtypeStruct((_N, 64), jnp.float32),
            jax.ShapeDtypeStruct((1, _F), jnp.float32),
            jax.ShapeDtypeStruct((_N, _K), jnp.int32),
            jax.ShapeDtypeStruct((_N, _K), jnp.float32),
        ],
        scratch_shapes=[
            pltpu.VMEM((_N, _F), jnp.float32),
            pltpu.VMEM((_N, 1), jnp.float32),
            pltpu.VMEM((1, _N), jnp.float32),
        ],
        compiler_params=pltpu.CompilerParams(
            dimension_semantics=("arbitrary",)),
    )(pts, w1t, b1r, w2t, b2r, w3t, b3r)

    pts_pad = jnp.pad(pts, ((0, 0), (0, 13)))
    knn_rows = _make_sc_gather()(pts_pad, idx.reshape(_N * _K))
    knn48 = knn_rows.reshape(_N, _K * 16)

    out = pl.pallas_call(
        _stage_b2,
        grid=(nb,),
        in_specs=[
            pl.BlockSpec((_TB, 64), lambda i: (i, 0)),
            pl.BlockSpec((_TB, _K * 16), lambda i: (i, 0)),
            pl.BlockSpec((_TB, _K), lambda i: (i, 0)),
            pl.BlockSpec((_TB, 3), lambda i: (i, 0)),
            pl.BlockSpec((1, _F), lambda i: (0, 0)),
            pl.BlockSpec((_F, 512), lambda i: (0, 0)),
            pl.BlockSpec((1, 512), lambda i: (0, 0)),
            pl.BlockSpec((64, 512), lambda i: (0, 0)),
            pl.BlockSpec((12, 512), lambda i: (0, 0)),
            pl.BlockSpec((512, 256), lambda i: (0, 0)),
            pl.BlockSpec((1, 256), lambda i: (0, 0)),
            pl.BlockSpec((256, 6), lambda i: (0, 0)),
            pl.BlockSpec((1, 6), lambda i: (0, 0)),
        ],
        out_specs=pl.BlockSpec((_TB, 6), lambda i: (i, 0)),
        out_shape=jax.ShapeDtypeStruct((_N, 6), jnp.float32),
        scratch_shapes=[pltpu.VMEM((1, 512), jnp.float32)],
        compiler_params=pltpu.CompilerParams(
            dimension_semantics=("arbitrary",)),
    )(feat, knn48, dk, pts, gmax, w4at, b4r, w4bt, w4ct, w5t, b5r,
      w6t, b6r)
    return out


# speedup vs baseline: 1.0456x; 1.0456x over previous
import functools

import jax
import jax.numpy as jnp
from jax.experimental import pallas as pl
from jax.experimental.pallas import tpu as pltpu
from jax.experimental.pallas import tpu_sc as plsc

_N = 4096
_F = 1024
_TA = 1024
_TB = 512
_K = 3

_P = jax.lax.Precision.DEFAULT
_HI = jax.lax.Precision.HIGHEST


def _stage_ab1(p_ref, w1t_ref, b1_ref, w2t_ref, b2_ref, w3t_ref, b3_ref,
               feat_ref, gmax_ref, idx_ref, dk_ref, x_s, xxc_s, xxt_s):
    i = pl.program_id(0)

    @pl.when(i == 0)
    def _():
        ones = jnp.ones((1, _F), jnp.float32)
        for c in range(_N // _TA):
            sl = pl.ds(c * _TA, _TA)
            p = p_ref[sl, :]
            f1 = jax.nn.relu(
                jnp.dot(p, w1t_ref[...], precision=_P) + b1_ref[...])
            f = jax.nn.relu(
                jnp.dot(f1, w2t_ref[...], precision=_P) + b2_ref[...])
            f2 = jax.nn.relu(
                jnp.dot(f, w3t_ref[...], precision=_P) + b3_ref[...])
            feat_ref[sl, :] = f
            x_s[sl, :] = f2
            y = f2 * f2
            xxc_s[sl, :] = jnp.sum(y, axis=1, keepdims=True)
            xxt_s[:, sl] = jax.lax.dot_general(
                ones, y, (((1,), (1,)), ((), ())), precision=_HI)
            m = jnp.max(f2, axis=0, keepdims=True)
            if c == 0:
                gmax_ref[...] = m
            else:
                gmax_ref[...] = jnp.maximum(gmax_ref[...], m)

    r0 = i * _TB
    x_t2 = x_s[pl.ds(r0, _TB), :] * jnp.float32(-2.0)
    dots2 = jax.lax.dot_general(
        x_t2, x_s[...], (((1,), (1,)), ((), ())), precision=_P)
    t = (xxc_s[pl.ds(r0, _TB), :] + dots2) + xxt_s[...]

    iota = jax.lax.broadcasted_iota(jnp.int32, (_TB, _N), 1)
    vals = t
    idxs = []
    negd = []
    for r in range(_K + 1):
        m = jnp.max(vals, axis=1, keepdims=True)
        selmask = vals == m
        if r > 0:
            idxr = jnp.min(
                jnp.where(selmask, iota, _N), axis=1, keepdims=True)
            idxs.append(idxr)
            negd.append(-m)
        if r < _K:
            vals = jnp.where(selmask, jnp.float32(-1e30), vals)
    idx_ref[...] = jnp.concatenate(idxs, axis=1)
    dk_ref[...] = jnp.concatenate(negd, axis=1)


def _make_sc_gather():
    nc, ns = 2, 16
    nw = nc * ns
    b_total = _N * _K
    b_per_w = b_total // nw
    mesh = plsc.VectorSubcoreMesh(
        core_axis_name="c", subcore_axis_name="s",
        num_cores=nc, num_subcores=ns)

    @functools.partial(
        pl.kernel, mesh=mesh,
        out_type=jax.ShapeDtypeStruct((b_total, 16), jnp.float32),
        compiler_params=pltpu.CompilerParams(use_tc_tiling_on_sc=False),
        scratch_types=[
            pltpu.VMEM((b_per_w,), jnp.int32),
            pltpu.VMEM((b_per_w, 16), jnp.float32),
            pltpu.SemaphoreType.DMA,
        ],
    )
    def sc_gather(table_hbm, idx_hbm, out_hbm, idx_v, rows_v, sem):
        wid = jax.lax.axis_index("s") * nc + jax.lax.axis_index("c")
        base = wid * b_per_w
        pltpu.sync_copy(idx_hbm.at[pl.ds(base, b_per_w)], idx_v)
        copies = [
            pltpu.async_copy(
                table_hbm.at[idx_v.at[pl.ds(j * 128, 128)]],
                rows_v.at[pl.ds(j * 128, 128), :], sem)
            for j in range(b_per_w // 128)
        ]
        for c in copies:
            c.wait()
        pltpu.sync_copy(rows_v, out_hbm.at[pl.ds(base, b_per_w)])

    return sc_gather


def _stage_b2(feat_ref, knn_ref, dk_ref, pts_ref, gmax_ref,
              w4at_ref, b4_ref, w4bt_ref, w4ct_ref, w5t_ref, b5_ref,
              w6t_ref, b6_ref, out_ref, b4eff_ref):
    i = pl.program_id(0)

    @pl.when(i == 0)
    def _():
        b4eff_ref[...] = (
            jnp.dot(gmax_ref[...], w4at_ref[...], precision=_HI)
            + b4_ref[...])

    p_t = pts_ref[...]
    parts = []
    for r in range(_K):
        rel = knn_ref[:, r * 16:r * 16 + 3] - p_t
        parts.append(rel)
        parts.append(dk_ref[:, r:r + 1])
    knn12 = jnp.concatenate(parts, axis=1)

    h = jax.nn.relu(
        jnp.dot(feat_ref[...], w4bt_ref[...], precision=_P)
        + jnp.dot(knn12, w4ct_ref[...], precision=_HI)
        + b4eff_ref[...])
    h2 = jax.nn.relu(jnp.dot(h, w5t_ref[...], precision=_P) + b5_ref[...])
    logits = jnp.dot(h2, w6t_ref[...], precision=_P) + b6_ref[...]
    out_ref[...] = jax.nn.softmax(logits, axis=-1)


def kernel(points, W1, b1, W2, b2, W3, b3, W4, b4, W5, b5, W6, b6):
    pts = points.reshape(_N, 3)
    w1t, w2t, w3t = W1.T, W2.T, W3.T
    w4at = W4[:, :_F].T
    w4bt = W4[:, _F:_F + 64].T
    w4ct = W4[:, _F + 64:].T
    w5t, w6t = W5.T, W6.T
    b1r, b2r, b3r = b1.reshape(1, -1), b2.reshape(1, -1), b3.reshape(1, -1)
    b4r, b5r, b6r = b4.reshape(1, -1), b5.reshape(1, -1), b6.reshape(1, -1)

    nb = _N // _TB
    feat, gmax, idx, dk = pl.pallas_call(
        _stage_ab1,
        grid=(nb,),
        in_specs=[
            pl.BlockSpec((_N, 3), lambda i: (0, 0)),
            pl.BlockSpec((3, 64), lambda i: (0, 0)),
            pl.BlockSpec((1, 64), lambda i: (0, 0)),
            pl.BlockSpec((64, 64), lambda i: (0, 0)),
            pl.BlockSpec((1, 64), lambda i: (0, 0)),
            pl.BlockSpec((64, _F), lambda i: (0, 0)),
            pl.BlockSpec((1, _F), lambda i: (0, 0)),
        ],
        out_specs=[
            pl.BlockSpec((_N, 64), lambda i: (0, 0)),
            pl.BlockSpec((1, _F), lambda i: (0, 0)),
            pl.BlockSpec((_TB, _K), lambda i: (i, 0)),
            pl.BlockSpec((_TB, _K), lambda i: (i, 0)),
        ],
        out_shape=[
            jax.ShapeDtypeStruct((_N, 64), jnp.float32),
            jax.ShapeDtypeStruct((1, _F), jnp.float32),
            jax.ShapeDtypeStruct((_N, _K), jnp.int32),
            jax.ShapeDtypeStruct((_N, _K), jnp.float32),
        ],
        scratch_shapes=[
            pltpu.VMEM((_N, _F), jnp.float32),
            pltpu.VMEM((_N, 1), jnp.float32),
            pltpu.VMEM((1, _N), jnp.float32),
        ],
        compiler_params=pltpu.CompilerParams(
            dimension_semantics=("arbitrary",)),
    )(pts, w1t, b1r, w2t, b2r, w3t, b3r)

    pts_pad = jnp.pad(pts, ((0, 0), (0, 13)))
    knn_rows = _make_sc_gather()(pts_pad, idx.reshape(_N * _K))
    knn48 = knn_rows.reshape(_N, _K * 16)

    out = pl.pallas_call(
        _stage_b2,
        grid=(nb,),
        in_specs=[
            pl.BlockSpec((_TB, 64), lambda i: (i, 0)),
            pl.BlockSpec((_TB, _K * 16), lambda i: (i, 0)),
            pl.BlockSpec((_TB, _K), lambda i: (i, 0)),
            pl.BlockSpec((_TB, 3), lambda i: (i, 0)),
            pl.BlockSpec((1, _F), lambda i: (0, 0)),
            pl.BlockSpec((_F, 512), lambda i: (0, 0)),
            pl.BlockSpec((1, 512), lambda i: (0, 0)),
            pl.BlockSpec((64, 512), lambda i: (0, 0)),
            pl.BlockSpec((12, 512), lambda i: (0, 0)),
            pl.BlockSpec((512, 256), lambda i: (0, 0)),
            pl.BlockSpec((1, 256), lambda i: (0, 0)),
            pl.BlockSpec((256, 6), lambda i: (0, 0)),
            pl.BlockSpec((1, 6), lambda i: (0, 0)),
        ],
        out_specs=pl.BlockSpec((_TB, 6), lambda i: (i, 0)),
        out_shape=jax.ShapeDtypeStruct((_N, 6), jnp.float32),
        scratch_shapes=[pltpu.VMEM((1, 512), jnp.float32)],
        compiler_params=pltpu.CompilerParams(
            dimension_semantics=("arbitrary",)),
    )(feat, knn48, dk, pts, gmax, w4at, b4r, w4bt, w4ct, w5t, b5r,
      w6t, b6r)
    return out
